# baseline (speedup 1.0000x reference)
import functools

import jax
import jax.numpy as jnp
from jax import lax
from jax.experimental import pallas as pl
from jax.experimental.pallas import tpu as pltpu
from jax.experimental.pallas import tpu_sc as plsc

_B, _S, _D, _T = 16, 512, 256, 2048
_NC, _NS = 2, 16
_CH = 32
_NCH = (_T // 2) // _CH
_ZR = 32
_L = 16
_SENT = 2**31 - 1
_NSLOT = 8


def _sc_expand_body(x_hbm, dur_hbm, out_hbm,
                    dur_v, cum_v, mark_v, gidx_v,
                    bufa_v, bufb_v, bufc_v, bufd_v, bufe_v, buff_v,
                    bufg_v, bufh_v, zbuf_v,
                    gsem0, gsem1, gsem2, gsem3, gsem4, gsem5, gsem6, gsem7,
                    ssem0, ssem1, ssem2, ssem3, ssem4, ssem5, ssem6, ssem7,
                    zsem):
    cid = lax.axis_index("c")
    sid = lax.axis_index("s")
    wid = sid * _NC + cid
    b = wid // 2
    h = wid % 2
    iota = lax.iota(jnp.int32, _L)

    pltpu.sync_copy(dur_hbm.at[b], dur_v)

    def cum_body(i, carry):
        v = dur_v[pl.ds(i * _L, _L)]
        c = plsc.cumsum(v) + carry
        cum_v[pl.ds(i * _L, _L)] = c
        return c[_L - 1]

    total = lax.fori_loop(0, _S // _L, cum_body, jnp.int32(0))
    cum_v[pl.ds(_S, _L)] = jnp.full((_L,), _SENT, jnp.int32)

    row0 = b * _T
    bufs = (bufa_v, bufb_v, bufc_v, bufd_v, bufe_v, buff_v, bufg_v, bufh_v)
    gsems = (gsem0, gsem1, gsem2, gsem3, gsem4, gsem5, gsem6, gsem7)
    ssems = (ssem0, ssem1, ssem2, ssem3, ssem4, ssem5, ssem6, ssem7)
    par = jnp.bitwise_xor(h, b % 2)
    cgs = [2 * c + par for c in range(_NCH)]
    starts = [cg * _CH for cg in cgs]
    preds = [total > s for s in starts]
    nvals = [jnp.clip(total - s, 0, _CH) for s in starts]
    gds, sds, zds = [], [], []
    for c in range(_NCH):
        slot = c % _NSLOT
        gds.append(pltpu.make_async_copy(
            x_hbm.at[gidx_v.at[cgs[c]]], bufs[slot], gsems[slot]))
        sds.append(pltpu.make_async_copy(
            bufs[slot], out_hbm.at[pl.ds(row0 + starts[c], _CH)],
            ssems[slot]))
        zds.append(tuple(
            pltpu.make_async_copy(
                zbuf_v, out_hbm.at[pl.ds(row0 + starts[c] + z * _ZR, _ZR)],
                zsem)
            for z in range(_CH // _ZR)))

    def zbuf_body(r, _):
        for k in range(_D // _L):
            zbuf_v[r, pl.ds(k * _L, _L)] = jnp.zeros((_L,), jnp.float32)
        return 0

    lax.fori_loop(0, _ZR, zbuf_body, 0, unroll=4)

    for c in range(_NCH):
        @pl.when(jnp.logical_not(preds[c]))
        def _(c=c):
            for zd in zds[c]:
                zd.start()

    def zmark_body(i, _):
        mark_v[pl.ds(i * _L, _L)] = jnp.zeros((_L,), jnp.int32)
        return 0

    lax.fori_loop(0, _T // _L, zmark_body, 0, unroll=8)

    def mark_body(i, _):
        cur = cum_v[pl.ds(i * _L, _L)]
        nxt = plsc.load_gather(cum_v, [i * _L + 1 + iota])
        msk = (cur != nxt) & (cur < _T)
        plsc.store_scatter(mark_v, [jnp.minimum(cur, _T - 1)],
                           i * _L + 1 + iota, mask=msk)
        return 0

    lax.fori_loop(0, _S // _L, mark_body, 0)

    def idx_body(i, carry):
        m = jnp.maximum(plsc.cummax(mark_v[pl.ds(i * _L, _L)]), carry)
        gidx_v[i // (_CH // _L), pl.ds((i % (_CH // _L)) * _L, _L)] = (
            b * _S + jnp.minimum(m, _S - 1))
        rl = _CH // _L
        for ec in range(_NSLOT):
            @pl.when((i == (2 * ec + 1) * rl - 1 + par * rl) & preds[ec])
            def _(ec=ec):
                gds[ec].start()

        return m[_L - 1]

    lax.fori_loop(0, _T // _L, idx_body, jnp.int32(0), unroll=4)

    for c in range(_NCH):
        if _NSLOT - 1 <= c < _NCH - 1:
            @pl.when(preds[c - (_NSLOT - 1)])
            def _(c=c):
                sds[c - (_NSLOT - 1)].wait()

            @pl.when(preds[c + 1])
            def _(c=c):
                gds[c + 1].start()

        @pl.when(preds[c])
        def _(c=c):
            gds[c].wait()

            def zrow_body(r, _, buf=bufs[c % _NSLOT]):
                for k in range(_D // _L):
                    buf[r, pl.ds(k * _L, _L)] = jnp.zeros((_L,), jnp.float32)
                return 0

            lax.fori_loop(nvals[c], _CH, zrow_body, 0)
            sds[c].start()

    for c in range(_NCH - _NSLOT, _NCH):
        @pl.when(preds[c])
        def _(c=c):
            sds[c].wait()

    for c in range(_NCH):
        @pl.when(jnp.logical_not(preds[c]))
        def _(c=c):
            for zd in zds[c]:
                zd.wait()


@functools.partial(
    pl.kernel,
    out_type=jax.ShapeDtypeStruct((_B * _T, _D), jnp.float32),
    mesh=plsc.VectorSubcoreMesh(core_axis_name="c", subcore_axis_name="s"),
    scratch_types=(
        pltpu.VMEM((_S,), jnp.int32),
        pltpu.VMEM((_S + _L,), jnp.int32),
        pltpu.VMEM((_T,), jnp.int32),
        pltpu.VMEM((_T // _CH, _CH), jnp.int32),
        pltpu.VMEM((_CH, _D), jnp.float32),
        pltpu.VMEM((_CH, _D), jnp.float32),
        pltpu.VMEM((_CH, _D), jnp.float32),
        pltpu.VMEM((_CH, _D), jnp.float32),
        pltpu.VMEM((_CH, _D), jnp.float32),
        pltpu.VMEM((_CH, _D), jnp.float32),
        pltpu.VMEM((_CH, _D), jnp.float32),
        pltpu.VMEM((_CH, _D), jnp.float32),
        pltpu.VMEM((_ZR, _D), jnp.float32),
        pltpu.SemaphoreType.DMA,
        pltpu.SemaphoreType.DMA,
        pltpu.SemaphoreType.DMA,
        pltpu.SemaphoreType.DMA,
        pltpu.SemaphoreType.DMA,
        pltpu.SemaphoreType.DMA,
        pltpu.SemaphoreType.DMA,
        pltpu.SemaphoreType.DMA,
        pltpu.SemaphoreType.DMA,
        pltpu.SemaphoreType.DMA,
        pltpu.SemaphoreType.DMA,
        pltpu.SemaphoreType.DMA,
        pltpu.SemaphoreType.DMA,
        pltpu.SemaphoreType.DMA,
        pltpu.SemaphoreType.DMA,
        pltpu.SemaphoreType.DMA,
        pltpu.SemaphoreType.DMA,
    ),
    compiler_params=pltpu.CompilerParams(needs_layout_passes=False),
)
def _sc_expand(x_hbm, dur_hbm, out_hbm, *scratch):
    _sc_expand_body(x_hbm, dur_hbm, out_hbm, *scratch)


def _ln(x, g, bb):
    m = jnp.mean(x, axis=-1, keepdims=True)
    v = jnp.mean((x - m) * (x - m), axis=-1, keepdims=True)
    return (x - m) * lax.rsqrt(v + 1e-5) * g + bb


_GB = 4
_R = _GB * _S


def _pred_body(lip_ref, mask_ref, dur_ref, ml_ref, W1_ref, b1_ref, g1_ref,
               be1_ref, W2_ref, b2_ref, g2_ref, be2_ref, Wc_ref, bc_ref,
               out_ref, mel_ref):
    i = pl.program_id(0)

    @pl.when(i == 0)
    def _():
        totals = jnp.sum(dur_ref[...], axis=1)
        mel_ref[...] = jnp.minimum(totals, ml_ref[0, 0])

    hm = lip_ref[...]
    a = lax.dot_general(hm, W1_ref[...], (((1,), (1,)), ((), ())),
                        preferred_element_type=jnp.float32) + b1_ref[...]
    a = jnp.maximum(a, 0.0)
    a = _ln(a, g1_ref[...], be1_ref[...])
    a = lax.dot_general(a, W2_ref[...], (((1,), (1,)), ((), ())),
                        preferred_element_type=jnp.float32) + b2_ref[...]
    a = jnp.maximum(a, 0.0)
    a = _ln(a, g2_ref[...], be2_ref[...])
    o = lax.dot_general(Wc_ref[...], a, (((1,), (1,)), ((), ())),
                        preferred_element_type=jnp.float32)
    o = o + bc_ref[0, 0]
    o = jnp.logaddexp(o, 0.0)
    out_ref[0] = o * (1.0 - mask_ref[0])


def _predictor(lip2, mask3, dur2, ml2, W1, b1, g1, be1, W2, b2, g2, be2,
               Wc, bc2):
    wspec2 = pl.BlockSpec((_D, _D), lambda i: (0, 0))
    vspec = pl.BlockSpec((_D,), lambda i: (0,))
    return pl.pallas_call(
        _pred_body,
        grid=(_B // _GB,),
        in_specs=[
            pl.BlockSpec((_R, _D), lambda i: (i, 0)),
            pl.BlockSpec((1, 1, _R), lambda i: (i, 0, 0)),
            pl.BlockSpec((_B, _S), lambda i: (0, 0)),
            pl.BlockSpec((1, _D), lambda i: (0, 0)),
            wspec2, vspec, vspec, vspec,
            wspec2, vspec, vspec, vspec,
            pl.BlockSpec((1, _D), lambda i: (0, 0)),
            pl.BlockSpec((1, _D), lambda i: (0, 0)),
        ],
        out_specs=[
            pl.BlockSpec((1, 1, _R), lambda i: (i, 0, 0)),
            pl.BlockSpec((_B,), lambda i: (0,)),
        ],
        out_shape=[
            jax.ShapeDtypeStruct((_B // _GB, 1, _R), jnp.float32),
            jax.ShapeDtypeStruct((_B,), jnp.int32),
        ],
    )(lip2, mask3, dur2, ml2, W1, b1, g1, be1, W2, b2, g2, be2, Wc, bc2)


def kernel(x, output_text_lip, src_mask, duration_target, max_len,
           W1, b1, g1, be1, W2, b2, g2, be2, Wc, bc):
    x_flat = x.reshape(_B * _S, _D)
    dur = duration_target.astype(jnp.int32)
    out_flat = _sc_expand(x_flat, dur)

    lip2 = output_text_lip.reshape(_B * _S, _D)
    mask3 = src_mask.astype(jnp.float32).reshape(_B // _GB, 1, _R)
    bc2 = jnp.broadcast_to(bc.reshape(1, 1), (1, _D))
    ml2 = jnp.broadcast_to(
        jnp.asarray(max_len, jnp.int32).reshape(1, 1), (1, _D))
    log_dur3, mel_len = _predictor(lip2, mask3, dur, ml2,
                                   W1, b1, g1, be1, W2, b2, g2, be2, Wc, bc2)
    log_dur = log_dur3.reshape(_B, _S)

    x_expanded = out_flat.reshape(_B, _T, _D)
    return x_expanded, log_dur, duration_target, mel_len


# speedup vs baseline: 1.1229x; 1.1229x over previous
import functools

import jax
import jax.numpy as jnp
from jax import lax
from jax.experimental import pallas as pl
from jax.experimental.pallas import tpu as pltpu
from jax.experimental.pallas import tpu_sc as plsc

_B, _S, _D, _T = 16, 512, 256, 2048
_NC, _NS = 2, 16
_CH = 64
_NCH = (_T // 2) // _CH
_ZR = 64
_L = 16
_SENT = 2**31 - 1
_NSLOT = 6


def _sc_expand_body(x_hbm, dur_hbm, out_hbm,
                    dur_v, cum_v, mark_v, gidx_v,
                    bufa_v, bufb_v, bufc_v, bufd_v, bufe_v, buff_v, zbuf_v,
                    gsem0, gsem1, gsem2, gsem3, gsem4, gsem5,
                    ssem0, ssem1, ssem2, ssem3, ssem4, ssem5, zsem):
    cid = lax.axis_index("c")
    sid = lax.axis_index("s")
    wid = sid * _NC + cid
    b = wid // 2
    h = wid % 2
    iota = lax.iota(jnp.int32, _L)

    pltpu.sync_copy(dur_hbm.at[b], dur_v)

    def cum_body(i, carry):
        v = dur_v[pl.ds(i * _L, _L)]
        c = plsc.cumsum(v) + carry
        cum_v[pl.ds(i * _L, _L)] = c
        return c[_L - 1]

    total = lax.fori_loop(0, _S // _L, cum_body, jnp.int32(0))
    cum_v[pl.ds(_S, _L)] = jnp.full((_L,), _SENT, jnp.int32)

    row0 = b * _T
    bufs = (bufa_v, bufb_v, bufc_v, bufd_v, bufe_v, buff_v)
    gsems = (gsem0, gsem1, gsem2, gsem3, gsem4, gsem5)
    ssems = (ssem0, ssem1, ssem2, ssem3, ssem4, ssem5)
    par = jnp.bitwise_xor(h, b % 2)
    cgs = [2 * c + par for c in range(_NCH)]
    starts = [cg * _CH for cg in cgs]
    preds = [total > s for s in starts]
    nvals = [jnp.clip(total - s, 0, _CH) for s in starts]
    gds, sds, zds = [], [], []
    for c in range(_NCH):
        slot = c % _NSLOT
        gds.append(pltpu.make_async_copy(
            x_hbm.at[gidx_v.at[cgs[c]]], bufs[slot], gsems[slot]))
        sds.append(pltpu.make_async_copy(
            bufs[slot], out_hbm.at[pl.ds(row0 + starts[c], _CH)],
            ssems[slot]))
        zds.append(tuple(
            pltpu.make_async_copy(
                zbuf_v, out_hbm.at[pl.ds(row0 + starts[c] + z * _ZR, _ZR)],
                zsem)
            for z in range(_CH // _ZR)))

    def zbuf_body(r, _):
        for k in range(_D // _L):
            zbuf_v[r, pl.ds(k * _L, _L)] = jnp.zeros((_L,), jnp.float32)
        return 0

    lax.fori_loop(0, _ZR, zbuf_body, 0, unroll=4)

    for c in range(_NCH):
        @pl.when(jnp.logical_not(preds[c]))
        def _(c=c):
            for zd in zds[c]:
                zd.start()

    def zmark_body(i, _):
        mark_v[pl.ds(i * _L, _L)] = jnp.zeros((_L,), jnp.int32)
        return 0

    lax.fori_loop(0, _T // _L, zmark_body, 0, unroll=8)

    def mark_body(i, _):
        cur = cum_v[pl.ds(i * _L, _L)]
        nxt = plsc.load_gather(cum_v, [i * _L + 1 + iota])
        msk = (cur != nxt) & (cur < _T)
        plsc.store_scatter(mark_v, [jnp.minimum(cur, _T - 1)],
                           i * _L + 1 + iota, mask=msk)
        return 0

    lax.fori_loop(0, _S // _L, mark_body, 0)

    def idx_body(i, carry):
        m = jnp.maximum(plsc.cummax(mark_v[pl.ds(i * _L, _L)]), carry)
        gidx_v[i // (_CH // _L), pl.ds((i % (_CH // _L)) * _L, _L)] = (
            b * _S + jnp.minimum(m, _S - 1))
        rl = _CH // _L
        for ec in range(_NSLOT):
            @pl.when((i == (2 * ec + 1) * rl - 1 + par * rl) & preds[ec])
            def _(ec=ec):
                gds[ec].start()

        return m[_L - 1]

    lax.fori_loop(0, _T // _L, idx_body, jnp.int32(0), unroll=4)

    for c in range(_NCH):
        if _NSLOT - 1 <= c < _NCH - 1:
            @pl.when(preds[c - (_NSLOT - 1)])
            def _(c=c):
                sds[c - (_NSLOT - 1)].wait()

            @pl.when(preds[c + 1])
            def _(c=c):
                gds[c + 1].start()

        @pl.when(preds[c])
        def _(c=c):
            gds[c].wait()

            def zrow_body(r, _, buf=bufs[c % _NSLOT]):
                for k in range(_D // _L):
                    buf[r, pl.ds(k * _L, _L)] = jnp.zeros((_L,), jnp.float32)
                return 0

            lax.fori_loop(nvals[c], _CH, zrow_body, 0)
            sds[c].start()

    for c in range(_NCH - _NSLOT, _NCH):
        @pl.when(preds[c])
        def _(c=c):
            sds[c].wait()

    for c in range(_NCH):
        @pl.when(jnp.logical_not(preds[c]))
        def _(c=c):
            for zd in zds[c]:
                zd.wait()


@functools.partial(
    pl.kernel,
    out_type=jax.ShapeDtypeStruct((_B * _T, _D), jnp.float32),
    mesh=plsc.VectorSubcoreMesh(core_axis_name="c", subcore_axis_name="s"),
    scratch_types=(
        pltpu.VMEM((_S,), jnp.int32),
        pltpu.VMEM((_S + _L,), jnp.int32),
        pltpu.VMEM((_T,), jnp.int32),
        pltpu.VMEM((_T // _CH, _CH), jnp.int32),
        pltpu.VMEM((_CH, _D), jnp.float32),
        pltpu.VMEM((_CH, _D), jnp.float32),
        pltpu.VMEM((_CH, _D), jnp.float32),
        pltpu.VMEM((_CH, _D), jnp.float32),
        pltpu.VMEM((_CH, _D), jnp.float32),
        pltpu.VMEM((_CH, _D), jnp.float32),
        pltpu.VMEM((_ZR, _D), jnp.float32),
        pltpu.SemaphoreType.DMA,
        pltpu.SemaphoreType.DMA,
        pltpu.SemaphoreType.DMA,
        pltpu.SemaphoreType.DMA,
        pltpu.SemaphoreType.DMA,
        pltpu.SemaphoreType.DMA,
        pltpu.SemaphoreType.DMA,
        pltpu.SemaphoreType.DMA,
        pltpu.SemaphoreType.DMA,
        pltpu.SemaphoreType.DMA,
        pltpu.SemaphoreType.DMA,
        pltpu.SemaphoreType.DMA,
        pltpu.SemaphoreType.DMA,
    ),
    compiler_params=pltpu.CompilerParams(needs_layout_passes=False),
)
def _sc_expand(x_hbm, dur_hbm, out_hbm, *scratch):
    _sc_expand_body(x_hbm, dur_hbm, out_hbm, *scratch)


def _ln(x, g, bb):
    m = jnp.mean(x, axis=-1, keepdims=True)
    v = jnp.mean((x - m) * (x - m), axis=-1, keepdims=True)
    return (x - m) * lax.rsqrt(v + 1e-5) * g + bb


_GB = 4
_R = _GB * _S


def _pred_body(lip_ref, mask_ref, dur_ref, ml_ref, W1_ref, b1_ref, g1_ref,
               be1_ref, W2_ref, b2_ref, g2_ref, be2_ref, Wc_ref, bc_ref,
               out_ref, mel_ref):
    i = pl.program_id(0)

    @pl.when(i == 0)
    def _():
        totals = jnp.sum(dur_ref[...], axis=1)
        mel_ref[...] = jnp.minimum(totals, ml_ref[0, 0])

    hm = lip_ref[...]
    a = lax.dot_general(hm, W1_ref[...], (((1,), (1,)), ((), ())),
                        preferred_element_type=jnp.float32) + b1_ref[...]
    a = jnp.maximum(a, 0.0)
    a = _ln(a, g1_ref[...], be1_ref[...])
    a = lax.dot_general(a, W2_ref[...], (((1,), (1,)), ((), ())),
                        preferred_element_type=jnp.float32) + b2_ref[...]
    a = jnp.maximum(a, 0.0)
    a = _ln(a, g2_ref[...], be2_ref[...])
    o = lax.dot_general(Wc_ref[...], a, (((1,), (1,)), ((), ())),
                        preferred_element_type=jnp.float32)
    o = o + bc_ref[0, 0]
    o = jnp.logaddexp(o, 0.0)
    out_ref[0] = o * (1.0 - mask_ref[0])


def _predictor(lip2, mask3, dur2, ml2, W1, b1, g1, be1, W2, b2, g2, be2,
               Wc, bc2):
    wspec2 = pl.BlockSpec((_D, _D), lambda i: (0, 0))
    vspec = pl.BlockSpec((_D,), lambda i: (0,))
    return pl.pallas_call(
        _pred_body,
        grid=(_B // _GB,),
        in_specs=[
            pl.BlockSpec((_R, _D), lambda i: (i, 0)),
            pl.BlockSpec((1, 1, _R), lambda i: (i, 0, 0)),
            pl.BlockSpec((_B, _S), lambda i: (0, 0)),
            pl.BlockSpec((1, _D), lambda i: (0, 0)),
            wspec2, vspec, vspec, vspec,
            wspec2, vspec, vspec, vspec,
            pl.BlockSpec((1, _D), lambda i: (0, 0)),
            pl.BlockSpec((1, _D), lambda i: (0, 0)),
        ],
        out_specs=[
            pl.BlockSpec((1, 1, _R), lambda i: (i, 0, 0)),
            pl.BlockSpec((_B,), lambda i: (0,)),
        ],
        out_shape=[
            jax.ShapeDtypeStruct((_B // _GB, 1, _R), jnp.float32),
            jax.ShapeDtypeStruct((_B,), jnp.int32),
        ],
    )(lip2, mask3, dur2, ml2, W1, b1, g1, be1, W2, b2, g2, be2, Wc, bc2)


def kernel(x, output_text_lip, src_mask, duration_target, max_len,
           W1, b1, g1, be1, W2, b2, g2, be2, Wc, bc):
    x_flat = x.reshape(_B * _S, _D)
    dur = duration_target.astype(jnp.int32)
    out_flat = _sc_expand(x_flat, dur)

    lip2 = output_text_lip.reshape(_B * _S, _D)
    mask3 = src_mask.astype(jnp.float32).reshape(_B // _GB, 1, _R)
    bc2 = jnp.broadcast_to(bc.reshape(1, 1), (1, _D))
    ml2 = jnp.broadcast_to(
        jnp.asarray(max_len, jnp.int32).reshape(1, 1), (1, _D))
    log_dur3, mel_len = _predictor(lip2, mask3, dur, ml2,
                                   W1, b1, g1, be1, W2, b2, g2, be2, Wc, bc2)
    log_dur = log_dur3.reshape(_B, _S)

    x_expanded = out_flat.reshape(_B, _T, _D)
    return x_expanded, log_dur, duration_target, mel_len


# speedup vs baseline: 1.1274x; 1.0040x over previous
import functools

import jax
import jax.numpy as jnp
from jax import lax
from jax.experimental import pallas as pl
from jax.experimental.pallas import tpu as pltpu
from jax.experimental.pallas import tpu_sc as plsc

_B, _S, _D, _T = 16, 512, 256, 2048
_NC, _NS = 2, 16
_CH = 64
_NCH = (_T // 2) // _CH
_ZR = 64
_L = 16
_SENT = 2**31 - 1
_NSLOT = 5


def _sc_expand_body(x_hbm, dur_hbm, out_hbm,
                    dur_v, cum_v, mark_v, gidx_v,
                    bufa_v, bufb_v, bufc_v, bufd_v, bufe_v, zbuf_v,
                    gsem0, gsem1, gsem2, gsem3, gsem4,
                    ssem0, ssem1, ssem2, ssem3, ssem4, zsem):
    cid = lax.axis_index("c")
    sid = lax.axis_index("s")
    wid = sid * _NC + cid
    b = wid // 2
    h = wid % 2
    iota = lax.iota(jnp.int32, _L)

    pltpu.sync_copy(dur_hbm.at[b], dur_v)

    def cum_body(i, carry):
        v = dur_v[pl.ds(i * _L, _L)]
        c = plsc.cumsum(v) + carry
        cum_v[pl.ds(i * _L, _L)] = c
        return c[_L - 1]

    total = lax.fori_loop(0, _S // _L, cum_body, jnp.int32(0))
    cum_v[pl.ds(_S, _L)] = jnp.full((_L,), _SENT, jnp.int32)

    row0 = b * _T
    bufs = (bufa_v, bufb_v, bufc_v, bufd_v, bufe_v)
    gsems = (gsem0, gsem1, gsem2, gsem3, gsem4)
    ssems = (ssem0, ssem1, ssem2, ssem3, ssem4)
    par = jnp.bitwise_xor(h, b % 2)
    cgs = [2 * c + par for c in range(_NCH)]
    starts = [cg * _CH for cg in cgs]
    preds = [total > s for s in starts]
    nvals = [jnp.clip(total - s, 0, _CH) for s in starts]
    gds, sds, zds = [], [], []
    for c in range(_NCH):
        slot = c % _NSLOT
        gds.append(pltpu.make_async_copy(
            x_hbm.at[gidx_v.at[cgs[c]]], bufs[slot], gsems[slot]))
        sds.append(pltpu.make_async_copy(
            bufs[slot], out_hbm.at[pl.ds(row0 + starts[c], _CH)],
            ssems[slot]))
        zds.append(tuple(
            pltpu.make_async_copy(
                zbuf_v, out_hbm.at[pl.ds(row0 + starts[c] + z * _ZR, _ZR)],
                zsem)
            for z in range(_CH // _ZR)))

    def zbuf_body(r, _):
        for k in range(_D // _L):
            zbuf_v[r, pl.ds(k * _L, _L)] = jnp.zeros((_L,), jnp.float32)
        return 0

    lax.fori_loop(0, _ZR, zbuf_body, 0, unroll=4)

    for c in range(_NCH):
        @pl.when(jnp.logical_not(preds[c]))
        def _(c=c):
            for zd in zds[c]:
                zd.start()

    def zmark_body(i, _):
        mark_v[pl.ds(i * _L, _L)] = jnp.zeros((_L,), jnp.int32)
        return 0

    lax.fori_loop(0, _T // _L, zmark_body, 0, unroll=8)

    def mark_body(i, _):
        cur = cum_v[pl.ds(i * _L, _L)]
        nxt = plsc.load_gather(cum_v, [i * _L + 1 + iota])
        msk = (cur != nxt) & (cur < _T)
        plsc.store_scatter(mark_v, [jnp.minimum(cur, _T - 1)],
                           i * _L + 1 + iota, mask=msk)
        return 0

    lax.fori_loop(0, _S // _L, mark_body, 0)

    def idx_body(i, carry):
        m = jnp.maximum(plsc.cummax(mark_v[pl.ds(i * _L, _L)]), carry)
        gidx_v[i // (_CH // _L), pl.ds((i % (_CH // _L)) * _L, _L)] = (
            b * _S + jnp.minimum(m, _S - 1))
        rl = _CH // _L
        for ec in range(_NSLOT):
            @pl.when((i == (2 * ec + 1) * rl - 1 + par * rl) & preds[ec])
            def _(ec=ec):
                gds[ec].start()

        return m[_L - 1]

    lax.fori_loop(0, _T // _L, idx_body, jnp.int32(0), unroll=4)

    for c in range(_NCH):
        if _NSLOT - 1 <= c < _NCH - 1:
            @pl.when(preds[c - (_NSLOT - 1)])
            def _(c=c):
                sds[c - (_NSLOT - 1)].wait()

            @pl.when(preds[c + 1])
            def _(c=c):
                gds[c + 1].start()

        @pl.when(preds[c])
        def _(c=c):
            gds[c].wait()

            def zrow_body(r, _, buf=bufs[c % _NSLOT]):
                for k in range(_D // _L):
                    buf[r, pl.ds(k * _L, _L)] = jnp.zeros((_L,), jnp.float32)
                return 0

            lax.fori_loop(nvals[c], _CH, zrow_body, 0)
            sds[c].start()

    for c in range(_NCH - _NSLOT, _NCH):
        @pl.when(preds[c])
        def _(c=c):
            sds[c].wait()

    for c in range(_NCH):
        @pl.when(jnp.logical_not(preds[c]))
        def _(c=c):
            for zd in zds[c]:
                zd.wait()


@functools.partial(
    pl.kernel,
    out_type=jax.ShapeDtypeStruct((_B * _T, _D), jnp.float32),
    mesh=plsc.VectorSubcoreMesh(core_axis_name="c", subcore_axis_name="s"),
    scratch_types=(
        pltpu.VMEM((_S,), jnp.int32),
        pltpu.VMEM((_S + _L,), jnp.int32),
        pltpu.VMEM((_T,), jnp.int32),
        pltpu.VMEM((_T // _CH, _CH), jnp.int32),
        pltpu.VMEM((_CH, _D), jnp.float32),
        pltpu.VMEM((_CH, _D), jnp.float32),
        pltpu.VMEM((_CH, _D), jnp.float32),
        pltpu.VMEM((_CH, _D), jnp.float32),
        pltpu.VMEM((_CH, _D), jnp.float32),
        pltpu.VMEM((_ZR, _D), jnp.float32),
        pltpu.SemaphoreType.DMA,
        pltpu.SemaphoreType.DMA,
        pltpu.SemaphoreType.DMA,
        pltpu.SemaphoreType.DMA,
        pltpu.SemaphoreType.DMA,
        pltpu.SemaphoreType.DMA,
        pltpu.SemaphoreType.DMA,
        pltpu.SemaphoreType.DMA,
        pltpu.SemaphoreType.DMA,
        pltpu.SemaphoreType.DMA,
        pltpu.SemaphoreType.DMA,
    ),
    compiler_params=pltpu.CompilerParams(needs_layout_passes=False),
)
def _sc_expand(x_hbm, dur_hbm, out_hbm, *scratch):
    _sc_expand_body(x_hbm, dur_hbm, out_hbm, *scratch)


def _ln(x, g, bb):
    m = jnp.mean(x, axis=-1, keepdims=True)
    v = jnp.mean((x - m) * (x - m), axis=-1, keepdims=True)
    return (x - m) * lax.rsqrt(v + 1e-5) * g + bb


_GB = 4
_R = _GB * _S


def _pred_body(lip_ref, mask_ref, dur_ref, ml_ref, W1_ref, b1_ref, g1_ref,
               be1_ref, W2_ref, b2_ref, g2_ref, be2_ref, Wc_ref, bc_ref,
               out_ref, mel_ref):
    i = pl.program_id(0)

    @pl.when(i == 0)
    def _():
        totals = jnp.sum(dur_ref[...], axis=1)
        mel_ref[...] = jnp.minimum(totals, ml_ref[0, 0])

    hm = lip_ref[...]
    a = lax.dot_general(hm, W1_ref[...], (((1,), (1,)), ((), ())),
                        preferred_element_type=jnp.float32) + b1_ref[...]
    a = jnp.maximum(a, 0.0)
    a = _ln(a, g1_ref[...], be1_ref[...])
    a = lax.dot_general(a, W2_ref[...], (((1,), (1,)), ((), ())),
                        preferred_element_type=jnp.float32) + b2_ref[...]
    a = jnp.maximum(a, 0.0)
    a = _ln(a, g2_ref[...], be2_ref[...])
    o = lax.dot_general(Wc_ref[...], a, (((1,), (1,)), ((), ())),
                        preferred_element_type=jnp.float32)
    o = o + bc_ref[0, 0]
    o = jnp.logaddexp(o, 0.0)
    out_ref[0] = o * (1.0 - mask_ref[0])


def _predictor(lip2, mask3, dur2, ml2, W1, b1, g1, be1, W2, b2, g2, be2,
               Wc, bc2):
    wspec2 = pl.BlockSpec((_D, _D), lambda i: (0, 0))
    vspec = pl.BlockSpec((_D,), lambda i: (0,))
    return pl.pallas_call(
        _pred_body,
        grid=(_B // _GB,),
        in_specs=[
            pl.BlockSpec((_R, _D), lambda i: (i, 0)),
            pl.BlockSpec((1, 1, _R), lambda i: (i, 0, 0)),
            pl.BlockSpec((_B, _S), lambda i: (0, 0)),
            pl.BlockSpec((1, _D), lambda i: (0, 0)),
            wspec2, vspec, vspec, vspec,
            wspec2, vspec, vspec, vspec,
            pl.BlockSpec((1, _D), lambda i: (0, 0)),
            pl.BlockSpec((1, _D), lambda i: (0, 0)),
        ],
        out_specs=[
            pl.BlockSpec((1, 1, _R), lambda i: (i, 0, 0)),
            pl.BlockSpec((_B,), lambda i: (0,)),
        ],
        out_shape=[
            jax.ShapeDtypeStruct((_B // _GB, 1, _R), jnp.float32),
            jax.ShapeDtypeStruct((_B,), jnp.int32),
        ],
    )(lip2, mask3, dur2, ml2, W1, b1, g1, be1, W2, b2, g2, be2, Wc, bc2)


def kernel(x, output_text_lip, src_mask, duration_target, max_len,
           W1, b1, g1, be1, W2, b2, g2, be2, Wc, bc):
    x_flat = x.reshape(_B * _S, _D)
    dur = duration_target.astype(jnp.int32)
    out_flat = _sc_expand(x_flat, dur)

    lip2 = output_text_lip.reshape(_B * _S, _D)
    mask3 = src_mask.astype(jnp.float32).reshape(_B // _GB, 1, _R)
    bc2 = jnp.broadcast_to(bc.reshape(1, 1), (1, _D))
    ml2 = jnp.broadcast_to(
        jnp.asarray(max_len, jnp.int32).reshape(1, 1), (1, _D))
    log_dur3, mel_len = _predictor(lip2, mask3, dur, ml2,
                                   W1, b1, g1, be1, W2, b2, g2, be2, Wc, bc2)
    log_dur = log_dur3.reshape(_B, _S)

    x_expanded = out_flat.reshape(_B, _T, _D)
    return x_expanded, log_dur, duration_target, mel_len
